# Initial kernel scaffold; baseline (speedup 1.0000x reference)
import functools

import jax
import jax.numpy as jnp
from jax import lax
from jax.experimental import pallas as pl
from jax.experimental.pallas import tpu as pltpu
from jax.experimental.pallas import tpu_sc as plsc

EMBED_DIM = 64
BATCH = 4096
SEQ = 200
NUM_SEGS = 2 * BATCH

NC = 2
NS = 16
NW = NC * NS
SEGS_PER_W = NUM_SEGS // NW


def _pool_body(x_hbm, table_hbm, out_hbm, idx_v, rows_v, acc_v, sem):
    wid = lax.axis_index("s") * NC + lax.axis_index("c")
    seg0 = wid * SEGS_PER_W

    def seg_loop(seg, _):
        base = (seg0 + seg) * SEQ
        pltpu.sync_copy(x_hbm.at[pl.ds(base, SEQ)], idx_v)
        pltpu.async_copy(table_hbm.at[idx_v], rows_v, sem).wait()

        zero = jnp.zeros((16,), jnp.float32)

        def red(j, carry):
            a0, a1, a2, a3 = carry
            a0 = a0 + rows_v[j, pl.ds(0, 16)]
            a1 = a1 + rows_v[j, pl.ds(16, 16)]
            a2 = a2 + rows_v[j, pl.ds(32, 16)]
            a3 = a3 + rows_v[j, pl.ds(48, 16)]
            return (a0, a1, a2, a3)

        a0, a1, a2, a3 = lax.fori_loop(0, SEQ, red, (zero, zero, zero, zero))
        acc_v[seg, pl.ds(0, 16)] = a0
        acc_v[seg, pl.ds(16, 16)] = a1
        acc_v[seg, pl.ds(32, 16)] = a2
        acc_v[seg, pl.ds(48, 16)] = a3
        return 0

    lax.fori_loop(0, SEGS_PER_W, seg_loop, 0)
    pltpu.sync_copy(acc_v, out_hbm.at[pl.ds(seg0, SEGS_PER_W)])


_pool = functools.partial(
    pl.kernel,
    out_type=jax.ShapeDtypeStruct((NUM_SEGS, EMBED_DIM), jnp.float32),
    mesh=plsc.VectorSubcoreMesh(core_axis_name="c", subcore_axis_name="s"),
    scratch_types=[
        pltpu.VMEM((SEQ,), jnp.int32),
        pltpu.VMEM((SEQ, EMBED_DIM), jnp.float32),
        pltpu.VMEM((SEGS_PER_W, EMBED_DIM), jnp.float32),
        pltpu.SemaphoreType.DMA,
    ],
)(_pool_body)


def _mlp_body(s1_ref, s2_ref, w1a_ref, w1b_ref, b1_ref, w2_ref, out_ref):
    h = (
        jnp.dot(s1_ref[...], w1a_ref[...], preferred_element_type=jnp.float32)
        + jnp.dot(s2_ref[...], w1b_ref[...], preferred_element_type=jnp.float32)
        + b1_ref[...]
    )
    h = jnp.maximum(h, 0.0)
    logits = jnp.dot(h, w2_ref[...], preferred_element_type=jnp.float32)
    m = jnp.max(logits, axis=-1, keepdims=True)
    e = jnp.exp(logits - m)
    out_ref[...] = e / jnp.sum(e, axis=-1, keepdims=True)


def _mlp(s1, s2, w1a, w1b, b1, w2):
    blk = 512
    grid = (BATCH // blk,)
    return pl.pallas_call(
        _mlp_body,
        grid=grid,
        in_specs=[
            pl.BlockSpec((blk, EMBED_DIM), lambda i: (i, 0)),
            pl.BlockSpec((blk, EMBED_DIM), lambda i: (i, 0)),
            pl.BlockSpec((EMBED_DIM, 128), lambda i: (0, 0)),
            pl.BlockSpec((EMBED_DIM, 128), lambda i: (0, 0)),
            pl.BlockSpec((1, 128), lambda i: (0, 0)),
            pl.BlockSpec((128, 2), lambda i: (0, 0)),
        ],
        out_specs=pl.BlockSpec((blk, 2), lambda i: (i, 0)),
        out_shape=jax.ShapeDtypeStruct((BATCH, 2), jnp.float32),
    )(s1, s2, w1a, w1b, b1, w2)


def kernel(x, table, W1, b1, W2):
    x_flat = x.reshape(NUM_SEGS * SEQ)
    pooled = _pool(x_flat, table)
    s1 = pooled[:BATCH]
    s2 = pooled[BATCH:]
    w1a = W1[:EMBED_DIM]
    w1b = W1[EMBED_DIM:]
    return _mlp(s1, s2, w1a, w1b, b1.reshape(1, 128), W2)


# speedup vs baseline: 1.1465x; 1.1465x over previous
import functools

import jax
import jax.numpy as jnp
from jax import lax
from jax.experimental import pallas as pl
from jax.experimental.pallas import tpu as pltpu
from jax.experimental.pallas import tpu_sc as plsc

EMBED_DIM = 64
BATCH = 4096
SEQ = 200
NUM_SEGS = 2 * BATCH

NC = 2
NS = 16
NW = NC * NS
SEGS_PER_W = NUM_SEGS // NW


def _pool_body(x_hbm, table_hbm, out_hbm, idx_v, rows_v, acc_v, sem):
    wid = lax.axis_index("s") * NC + lax.axis_index("c")
    seg0 = wid * SEGS_PER_W

    def seg_loop(seg, _):
        base = (seg0 + seg) * SEQ
        pltpu.sync_copy(x_hbm.at[pl.ds(base, SEQ)], idx_v)
        pltpu.async_copy(table_hbm.at[idx_v], rows_v, sem).wait()

        zero = jnp.zeros((16,), jnp.float32)

        def red(j, carry):
            a0, a1, a2, a3 = carry
            a0 = a0 + rows_v[j, pl.ds(0, 16)]
            a1 = a1 + rows_v[j, pl.ds(16, 16)]
            a2 = a2 + rows_v[j, pl.ds(32, 16)]
            a3 = a3 + rows_v[j, pl.ds(48, 16)]
            return (a0, a1, a2, a3)

        a0, a1, a2, a3 = lax.fori_loop(0, SEQ, red, (zero, zero, zero, zero))
        acc_v[seg, pl.ds(0, 16)] = a0
        acc_v[seg, pl.ds(16, 16)] = a1
        acc_v[seg, pl.ds(32, 16)] = a2
        acc_v[seg, pl.ds(48, 16)] = a3
        return 0

    lax.fori_loop(0, SEGS_PER_W, seg_loop, 0)
    pltpu.sync_copy(acc_v, out_hbm.at[pl.ds(seg0, SEGS_PER_W)])


_pool = functools.partial(
    pl.kernel,
    out_type=jax.ShapeDtypeStruct((NUM_SEGS, EMBED_DIM), jnp.float32),
    mesh=plsc.VectorSubcoreMesh(core_axis_name="c", subcore_axis_name="s"),
    scratch_types=[
        pltpu.VMEM((SEQ,), jnp.int32),
        pltpu.VMEM((SEQ, EMBED_DIM), jnp.float32),
        pltpu.VMEM((SEGS_PER_W, EMBED_DIM), jnp.float32),
        pltpu.SemaphoreType.DMA,
    ],
    compiler_params=pltpu.CompilerParams(use_tc_tiling_on_sc=False),
)(_pool_body)


def _mlp_body(s1_ref, s2_ref, w1a_ref, w1b_ref, b1_ref, w2_ref, out_ref):
    h = (
        jnp.dot(s1_ref[...], w1a_ref[...], preferred_element_type=jnp.float32)
        + jnp.dot(s2_ref[...], w1b_ref[...], preferred_element_type=jnp.float32)
        + b1_ref[...]
    )
    h = jnp.maximum(h, 0.0)
    logits = jnp.dot(h, w2_ref[...], preferred_element_type=jnp.float32)
    m = jnp.max(logits, axis=-1, keepdims=True)
    e = jnp.exp(logits - m)
    out_ref[...] = e / jnp.sum(e, axis=-1, keepdims=True)


def _mlp(s1, s2, w1a, w1b, b1, w2):
    blk = 512
    grid = (BATCH // blk,)
    return pl.pallas_call(
        _mlp_body,
        grid=grid,
        in_specs=[
            pl.BlockSpec((blk, EMBED_DIM), lambda i: (i, 0)),
            pl.BlockSpec((blk, EMBED_DIM), lambda i: (i, 0)),
            pl.BlockSpec((EMBED_DIM, 128), lambda i: (0, 0)),
            pl.BlockSpec((EMBED_DIM, 128), lambda i: (0, 0)),
            pl.BlockSpec((1, 128), lambda i: (0, 0)),
            pl.BlockSpec((128, 2), lambda i: (0, 0)),
        ],
        out_specs=pl.BlockSpec((blk, 2), lambda i: (i, 0)),
        out_shape=jax.ShapeDtypeStruct((BATCH, 2), jnp.float32),
    )(s1, s2, w1a, w1b, b1, w2)


def kernel(x, table, W1, b1, W2):
    x_flat = x.reshape(NUM_SEGS * SEQ)
    pooled = _pool(x_flat, table)
    s1 = pooled[:BATCH]
    s2 = pooled[BATCH:]
    w1a = W1[:EMBED_DIM]
    w1b = W1[EMBED_DIM:]
    return _mlp(s1, s2, w1a, w1b, b1.reshape(1, 128), W2)


# speedup vs baseline: 1.4987x; 1.3071x over previous
import functools

import jax
import jax.numpy as jnp
from jax import lax
from jax.experimental import pallas as pl
from jax.experimental.pallas import tpu as pltpu
from jax.experimental.pallas import tpu_sc as plsc

EMBED_DIM = 64
BATCH = 4096
SEQ = 200
NUM_SEGS = 2 * BATCH

NC = 2
NS = 16
NW = NC * NS
SEGS_PER_W = NUM_SEGS // NW


def _pool_body(x_hbm, table_hbm, out_hbm, idx_v, rows_v, acc_v, sem_g, sem_i):
    wid = lax.axis_index("s") * NC + lax.axis_index("c")
    seg0 = wid * SEGS_PER_W

    def idx_start(seg, buf):
        base = (seg0 + seg) * SEQ
        pltpu.async_copy(x_hbm.at[pl.ds(base, SEQ)], idx_v.at[buf], sem_i)

    def idx_wait(seg, buf):
        base = (seg0 + seg) * SEQ
        pltpu.make_async_copy(
            x_hbm.at[pl.ds(base, SEQ)], idx_v.at[buf], sem_i
        ).wait()

    def gather_start(buf):
        pltpu.async_copy(table_hbm.at[idx_v.at[buf]], rows_v.at[buf], sem_g)

    def gather_wait(buf):
        pltpu.make_async_copy(
            table_hbm.at[idx_v.at[buf]], rows_v.at[buf], sem_g
        ).wait()

    idx_start(0, 0)
    idx_wait(0, 0)
    gather_start(0)
    idx_start(1, 1)

    def seg_loop(seg, _):
        b = lax.rem(seg, 2)
        nb = 1 - b
        gather_wait(b)

        @pl.when(seg + 1 < SEGS_PER_W)
        def _():
            idx_wait(seg + 1, nb)
            gather_start(nb)

        @pl.when(seg + 2 < SEGS_PER_W)
        def _():
            idx_start(seg + 2, b)

        zero = jnp.zeros((16,), jnp.float32)

        def red(j, carry):
            a0, a1, a2, a3 = carry
            for r in range(4):
                row = j * 4 + r
                a0 = a0 + rows_v[b, row, pl.ds(0, 16)]
                a1 = a1 + rows_v[b, row, pl.ds(16, 16)]
                a2 = a2 + rows_v[b, row, pl.ds(32, 16)]
                a3 = a3 + rows_v[b, row, pl.ds(48, 16)]
            return (a0, a1, a2, a3)

        a0, a1, a2, a3 = lax.fori_loop(
            0, SEQ // 4, red, (zero, zero, zero, zero)
        )
        acc_v[seg, pl.ds(0, 16)] = a0
        acc_v[seg, pl.ds(16, 16)] = a1
        acc_v[seg, pl.ds(32, 16)] = a2
        acc_v[seg, pl.ds(48, 16)] = a3
        return 0

    lax.fori_loop(0, SEGS_PER_W, seg_loop, 0)
    pltpu.sync_copy(acc_v, out_hbm.at[pl.ds(seg0, SEGS_PER_W)])


_pool = functools.partial(
    pl.kernel,
    out_type=jax.ShapeDtypeStruct((NUM_SEGS, EMBED_DIM), jnp.float32),
    mesh=plsc.VectorSubcoreMesh(core_axis_name="c", subcore_axis_name="s"),
    scratch_types=[
        pltpu.VMEM((2, SEQ), jnp.int32),
        pltpu.VMEM((2, SEQ, EMBED_DIM), jnp.float32),
        pltpu.VMEM((SEGS_PER_W, EMBED_DIM), jnp.float32),
        pltpu.SemaphoreType.DMA,
        pltpu.SemaphoreType.DMA,
    ],
    compiler_params=pltpu.CompilerParams(use_tc_tiling_on_sc=False),
)(_pool_body)


def _mlp_body(s1_ref, s2_ref, w1a_ref, w1b_ref, b1_ref, w2_ref, out_ref):
    h = (
        jnp.dot(s1_ref[...], w1a_ref[...], preferred_element_type=jnp.float32)
        + jnp.dot(s2_ref[...], w1b_ref[...], preferred_element_type=jnp.float32)
        + b1_ref[...]
    )
    h = jnp.maximum(h, 0.0)
    logits = jnp.dot(h, w2_ref[...], preferred_element_type=jnp.float32)
    m = jnp.max(logits, axis=-1, keepdims=True)
    e = jnp.exp(logits - m)
    out_ref[...] = e / jnp.sum(e, axis=-1, keepdims=True)


def _mlp(s1, s2, w1a, w1b, b1, w2):
    blk = 512
    grid = (BATCH // blk,)
    return pl.pallas_call(
        _mlp_body,
        grid=grid,
        in_specs=[
            pl.BlockSpec((blk, EMBED_DIM), lambda i: (i, 0)),
            pl.BlockSpec((blk, EMBED_DIM), lambda i: (i, 0)),
            pl.BlockSpec((EMBED_DIM, 128), lambda i: (0, 0)),
            pl.BlockSpec((EMBED_DIM, 128), lambda i: (0, 0)),
            pl.BlockSpec((1, 128), lambda i: (0, 0)),
            pl.BlockSpec((128, 2), lambda i: (0, 0)),
        ],
        out_specs=pl.BlockSpec((blk, 2), lambda i: (i, 0)),
        out_shape=jax.ShapeDtypeStruct((BATCH, 2), jnp.float32),
    )(s1, s2, w1a, w1b, b1, w2)


def kernel(x, table, W1, b1, W2):
    x_flat = x.reshape(NUM_SEGS * SEQ)
    pooled = _pool(x_flat, table)
    s1 = pooled[:BATCH]
    s2 = pooled[BATCH:]
    w1a = W1[:EMBED_DIM]
    w1b = W1[EMBED_DIM:]
    return _mlp(s1, s2, w1a, w1b, b1.reshape(1, 128), W2)
